# baseline (speedup 1.0000x reference)
import jax
import jax.numpy as jnp
from jax.experimental import pallas as pl

_NB = 1024
_CH = 4


def _fp_kernel(ut_ref, k_ref, uf_ref, kf_ref, w1_ref, b1_ref, w2_ref,
               b2_ref, w3_ref, b3_ref, out_ref):
    m = k_ref.shape[1]
    nb = ut_ref.shape[2]
    cw = nb // _CH
    f32 = jnp.float32
    bf16 = jnp.bfloat16

    kn = k_ref[0]
    kf = kf_ref[0]
    w1 = w1_ref[...]
    w2 = w2_ref[...]
    w3 = w3_ref[...]
    b1 = b1_ref[...]
    b2 = b2_ref[...]
    b3 = b3_ref[...]
    c2 = kf.shape[0]
    w1a = w1[:, :c2].astype(bf16)
    w1b = w1[:, c2:].astype(bf16)
    w2b = w2.astype(bf16)
    w3b = w3.astype(bf16)

    def build_sel(lo):
        ut = ut_ref[0, :, lo:lo + cw]
        d0 = ut[0:1, :] - kn[:, 0:1]
        d1 = ut[1:2, :] - kn[:, 1:2]
        d2c = ut[2:3, :] - kn[:, 2:3]
        d2 = d0 * d0 + d1 * d1 + d2c * d2c

        inf = jnp.float32(jnp.inf)
        zero = jnp.float32(0.0)
        v0 = jnp.min(d2, axis=0, keepdims=True)
        r0 = 1.0 / (v0 + 1e-8)
        e0 = d2 == v0
        selu = jnp.where(e0, r0, zero)
        m1 = jnp.where(e0, inf, d2)
        v1 = jnp.min(m1, axis=0, keepdims=True)
        r1 = 1.0 / (v1 + 1e-8)
        e1 = m1 == v1
        selu = jnp.where(e1, r1, selu)
        m2 = jnp.where(e1, inf, m1)
        v2 = jnp.min(m2, axis=0, keepdims=True)
        r2 = 1.0 / (v2 + 1e-8)
        selu = jnp.where(m2 == v2, r2, selu)
        return selu, 1.0 / (r0 + r1 + r2)

    def mlp(lo, selu, invn):
        interp = jax.lax.dot_general(kf, selu, (((1,), (0,)), ((), ())),
                                     preferred_element_type=f32)
        interp = interp * invn
        h = jax.lax.dot_general(w1a, interp.astype(bf16),
                                (((1,), (0,)), ((), ())),
                                preferred_element_type=f32)
        h += jax.lax.dot_general(w1b, uf_ref[0, :, lo:lo + cw].astype(bf16),
                                 (((1,), (0,)), ((), ())),
                                 preferred_element_type=f32)
        h = jnp.maximum(h + b1, 0.0)
        h = jax.lax.dot_general(w2b, h.astype(bf16),
                                (((1,), (0,)), ((), ())),
                                preferred_element_type=f32)
        h = jnp.maximum(h + b2, 0.0)
        h = jax.lax.dot_general(w3b, h.astype(bf16),
                                (((1,), (0,)), ((), ())),
                                preferred_element_type=f32)
        out_ref[0, :, lo:lo + cw] = jnp.maximum(h + b3, 0.0)

    prev = build_sel(0)
    for c in range(1, _CH):
        cur = build_sel(c * cw)
        mlp((c - 1) * cw, *prev)
        prev = cur
    mlp((_CH - 1) * cw, *prev)


@jax.jit
def kernel(unknown, known, unknow_feats, known_feats, W1, b1, W2, b2, W3, b3):
    B, n, _ = unknown.shape
    m = known.shape[1]
    C1 = unknow_feats.shape[1]
    C2 = known_feats.shape[1]
    O = W3.shape[0]
    nb = _NB
    grid = (B, n // nb)

    ut = jnp.transpose(unknown, (0, 2, 1))

    out = pl.pallas_call(
        _fp_kernel,
        grid=grid,
        in_specs=[
            pl.BlockSpec((1, 3, nb), lambda b, j: (b, 0, j)),
            pl.BlockSpec((1, m, 3), lambda b, j: (b, 0, 0)),
            pl.BlockSpec((1, C1, nb), lambda b, j: (b, 0, j)),
            pl.BlockSpec((1, C2, m), lambda b, j: (b, 0, 0)),
            pl.BlockSpec((W1.shape[0], W1.shape[1]), lambda b, j: (0, 0)),
            pl.BlockSpec((W1.shape[0], 1), lambda b, j: (0, 0)),
            pl.BlockSpec((W2.shape[0], W2.shape[1]), lambda b, j: (0, 0)),
            pl.BlockSpec((W2.shape[0], 1), lambda b, j: (0, 0)),
            pl.BlockSpec((W3.shape[0], W3.shape[1]), lambda b, j: (0, 0)),
            pl.BlockSpec((W3.shape[0], 1), lambda b, j: (0, 0)),
        ],
        out_specs=pl.BlockSpec((1, O, nb), lambda b, j: (b, 0, j)),
        out_shape=jax.ShapeDtypeStruct((B, O, n), jnp.float32),
    )(ut, known, unknow_feats, known_feats,
      W1, b1.reshape(-1, 1), W2, b2.reshape(-1, 1), W3, b3.reshape(-1, 1))
    return out


# speedup vs baseline: 1.4905x; 1.4905x over previous
import functools

import jax
import jax.numpy as jnp
from jax.experimental import pallas as pl
from jax.experimental.pallas import tpu as pltpu

_NB = 1024


def _fp_kernel(ut_ref, k_ref, uf_ref, kf_ref, w1_ref, b1_ref, w2_ref,
               b2_ref, w3_ref, b3_ref, out_ref):
    m = k_ref.shape[1]
    nb = ut_ref.shape[2]

    ut = ut_ref[0]
    kn = k_ref[0]

    d0 = ut[0:1, :] - kn[:, 0:1]
    d1 = ut[1:2, :] - kn[:, 1:2]
    d2c = ut[2:3, :] - kn[:, 2:3]
    d2 = d0 * d0 + d1 * d1 + d2c * d2c

    inf = jnp.float32(jnp.inf)
    zero = jnp.float32(0.0)
    v0 = jnp.min(d2, axis=0, keepdims=True)
    m1 = jnp.where(d2 == v0, inf, d2)
    v1 = jnp.min(m1, axis=0, keepdims=True)
    m2 = jnp.where(m1 == v1, inf, m1)
    v2 = jnp.min(m2, axis=0, keepdims=True)
    r0 = 1.0 / (v0 + 1e-8)
    r1 = 1.0 / (v1 + 1e-8)
    r2 = 1.0 / (v2 + 1e-8)
    selu = jnp.where(d2 <= v2, 1.0 / (d2 + 1e-8), zero)

    f32 = jnp.float32
    interp = jax.lax.dot_general(kf_ref[0], selu, (((1,), (0,)), ((), ())),
                                 preferred_element_type=f32)
    interp = interp * (1.0 / (r0 + r1 + r2))

    bf16 = jnp.bfloat16
    w1 = w1_ref[...]
    c2 = interp.shape[0]
    h = jax.lax.dot_general(w1[:, :c2].astype(bf16), interp.astype(bf16),
                            (((1,), (0,)), ((), ())),
                            preferred_element_type=f32)
    h += jax.lax.dot_general(w1[:, c2:].astype(bf16),
                             uf_ref[0].astype(bf16),
                             (((1,), (0,)), ((), ())),
                             preferred_element_type=f32)
    h = jnp.maximum(h + b1_ref[...], 0.0)
    h = jax.lax.dot_general(w2_ref[...].astype(bf16), h.astype(bf16),
                            (((1,), (0,)), ((), ())),
                            preferred_element_type=f32)
    h = jnp.maximum(h + b2_ref[...], 0.0)
    h = jax.lax.dot_general(w3_ref[...].astype(bf16), h.astype(bf16),
                            (((1,), (0,)), ((), ())),
                            preferred_element_type=f32)
    out_ref[0] = jnp.maximum(h + b3_ref[...], 0.0)


@jax.jit
def kernel(unknown, known, unknow_feats, known_feats, W1, b1, W2, b2, W3, b3):
    B, n, _ = unknown.shape
    m = known.shape[1]
    C1 = unknow_feats.shape[1]
    C2 = known_feats.shape[1]
    O = W3.shape[0]
    nb = _NB
    grid = (B, n // nb)

    ut = jnp.transpose(unknown, (0, 2, 1))

    out = pl.pallas_call(
        _fp_kernel,
        grid=grid,
        in_specs=[
            pl.BlockSpec((1, 3, nb), lambda b, j: (b, 0, j)),
            pl.BlockSpec((1, m, 3), lambda b, j: (b, 0, 0)),
            pl.BlockSpec((1, C1, nb), lambda b, j: (b, 0, j)),
            pl.BlockSpec((1, C2, m), lambda b, j: (b, 0, 0)),
            pl.BlockSpec((W1.shape[0], W1.shape[1]), lambda b, j: (0, 0)),
            pl.BlockSpec((W1.shape[0], 1), lambda b, j: (0, 0)),
            pl.BlockSpec((W2.shape[0], W2.shape[1]), lambda b, j: (0, 0)),
            pl.BlockSpec((W2.shape[0], 1), lambda b, j: (0, 0)),
            pl.BlockSpec((W3.shape[0], W3.shape[1]), lambda b, j: (0, 0)),
            pl.BlockSpec((W3.shape[0], 1), lambda b, j: (0, 0)),
        ],
        out_specs=pl.BlockSpec((1, O, nb), lambda b, j: (b, 0, j)),
        out_shape=jax.ShapeDtypeStruct((B, O, n), jnp.float32),
    )(ut, known, unknow_feats, known_feats,
      W1, b1.reshape(-1, 1), W2, b2.reshape(-1, 1), W3, b3.reshape(-1, 1))
    return out


# speedup vs baseline: 1.5212x; 1.0206x over previous
import functools

import jax
import jax.numpy as jnp
from jax.experimental import pallas as pl
from jax.experimental.pallas import tpu as pltpu

_NB = 2048


def _fp_kernel(ut_ref, k_ref, uf_ref, kf_ref, w1_ref, b1_ref, w2_ref,
               b2_ref, w3_ref, b3_ref, out_ref):
    m = k_ref.shape[1]
    nb = ut_ref.shape[2]

    ut = ut_ref[0]
    kn = k_ref[0]

    d0 = ut[0:1, :] - kn[:, 0:1]
    d1 = ut[1:2, :] - kn[:, 1:2]
    d2c = ut[2:3, :] - kn[:, 2:3]
    d2 = d0 * d0 + d1 * d1 + d2c * d2c

    inf = jnp.float32(jnp.inf)
    zero = jnp.float32(0.0)
    v0 = jnp.min(d2, axis=0, keepdims=True)
    m1 = jnp.where(d2 == v0, inf, d2)
    v1 = jnp.min(m1, axis=0, keepdims=True)
    m2 = jnp.where(m1 == v1, inf, m1)
    v2 = jnp.min(m2, axis=0, keepdims=True)
    r0 = 1.0 / (v0 + 1e-8)
    r1 = 1.0 / (v1 + 1e-8)
    r2 = 1.0 / (v2 + 1e-8)
    selu = jnp.where(d2 <= v2, 1.0 / (d2 + 1e-8), zero)

    f32 = jnp.float32
    interp = jax.lax.dot_general(kf_ref[0], selu, (((1,), (0,)), ((), ())),
                                 preferred_element_type=f32)
    interp = interp * (1.0 / (r0 + r1 + r2))

    bf16 = jnp.bfloat16
    w1 = w1_ref[...]
    c2 = interp.shape[0]
    h = jax.lax.dot_general(w1[:, :c2].astype(bf16), interp.astype(bf16),
                            (((1,), (0,)), ((), ())),
                            preferred_element_type=f32)
    h += jax.lax.dot_general(w1[:, c2:].astype(bf16),
                             uf_ref[0].astype(bf16),
                             (((1,), (0,)), ((), ())),
                             preferred_element_type=f32)
    h = jnp.maximum(h + b1_ref[...], 0.0)
    h = jax.lax.dot_general(w2_ref[...].astype(bf16), h.astype(bf16),
                            (((1,), (0,)), ((), ())),
                            preferred_element_type=f32)
    h = jnp.maximum(h + b2_ref[...], 0.0)
    h = jax.lax.dot_general(w3_ref[...].astype(bf16), h.astype(bf16),
                            (((1,), (0,)), ((), ())),
                            preferred_element_type=f32)
    out_ref[0] = jnp.maximum(h + b3_ref[...], 0.0)


@jax.jit
def kernel(unknown, known, unknow_feats, known_feats, W1, b1, W2, b2, W3, b3):
    B, n, _ = unknown.shape
    m = known.shape[1]
    C1 = unknow_feats.shape[1]
    C2 = known_feats.shape[1]
    O = W3.shape[0]
    nb = _NB
    grid = (B, n // nb)

    ut = jnp.transpose(unknown, (0, 2, 1))

    out = pl.pallas_call(
        _fp_kernel,
        grid=grid,
        in_specs=[
            pl.BlockSpec((1, 3, nb), lambda b, j: (b, 0, j)),
            pl.BlockSpec((1, m, 3), lambda b, j: (b, 0, 0)),
            pl.BlockSpec((1, C1, nb), lambda b, j: (b, 0, j)),
            pl.BlockSpec((1, C2, m), lambda b, j: (b, 0, 0)),
            pl.BlockSpec((W1.shape[0], W1.shape[1]), lambda b, j: (0, 0)),
            pl.BlockSpec((W1.shape[0], 1), lambda b, j: (0, 0)),
            pl.BlockSpec((W2.shape[0], W2.shape[1]), lambda b, j: (0, 0)),
            pl.BlockSpec((W2.shape[0], 1), lambda b, j: (0, 0)),
            pl.BlockSpec((W3.shape[0], W3.shape[1]), lambda b, j: (0, 0)),
            pl.BlockSpec((W3.shape[0], 1), lambda b, j: (0, 0)),
        ],
        out_specs=pl.BlockSpec((1, O, nb), lambda b, j: (b, 0, j)),
        out_shape=jax.ShapeDtypeStruct((B, O, n), jnp.float32),
    )(ut, known, unknow_feats, known_feats,
      W1, b1.reshape(-1, 1), W2, b2.reshape(-1, 1), W3, b3.reshape(-1, 1))
    return out


# speedup vs baseline: 1.5305x; 1.0061x over previous
import functools

import jax
import jax.numpy as jnp
from jax.experimental import pallas as pl
from jax.experimental.pallas import tpu as pltpu

_NB = 4096


def _fp_kernel(ut_ref, k_ref, uf_ref, kf_ref, w1_ref, b1_ref, w2_ref,
               b2_ref, w3_ref, b3_ref, out_ref):
    m = k_ref.shape[1]
    nb = ut_ref.shape[2]

    ut = ut_ref[0]
    kn = k_ref[0]

    d0 = ut[0:1, :] - kn[:, 0:1]
    d1 = ut[1:2, :] - kn[:, 1:2]
    d2c = ut[2:3, :] - kn[:, 2:3]
    d2 = d0 * d0 + d1 * d1 + d2c * d2c

    inf = jnp.float32(jnp.inf)
    zero = jnp.float32(0.0)
    v0 = jnp.min(d2, axis=0, keepdims=True)
    m1 = jnp.where(d2 == v0, inf, d2)
    v1 = jnp.min(m1, axis=0, keepdims=True)
    m2 = jnp.where(m1 == v1, inf, m1)
    v2 = jnp.min(m2, axis=0, keepdims=True)
    r0 = 1.0 / (v0 + 1e-8)
    r1 = 1.0 / (v1 + 1e-8)
    r2 = 1.0 / (v2 + 1e-8)
    selu = jnp.where(d2 <= v2, 1.0 / (d2 + 1e-8), zero)

    f32 = jnp.float32
    interp = jax.lax.dot_general(kf_ref[0], selu, (((1,), (0,)), ((), ())),
                                 preferred_element_type=f32)
    interp = interp * (1.0 / (r0 + r1 + r2))

    bf16 = jnp.bfloat16
    w1 = w1_ref[...]
    c2 = interp.shape[0]
    h = jax.lax.dot_general(w1[:, :c2].astype(bf16), interp.astype(bf16),
                            (((1,), (0,)), ((), ())),
                            preferred_element_type=f32)
    h += jax.lax.dot_general(w1[:, c2:].astype(bf16),
                             uf_ref[0].astype(bf16),
                             (((1,), (0,)), ((), ())),
                             preferred_element_type=f32)
    h = jnp.maximum(h + b1_ref[...], 0.0)
    h = jax.lax.dot_general(w2_ref[...].astype(bf16), h.astype(bf16),
                            (((1,), (0,)), ((), ())),
                            preferred_element_type=f32)
    h = jnp.maximum(h + b2_ref[...], 0.0)
    h = jax.lax.dot_general(w3_ref[...].astype(bf16), h.astype(bf16),
                            (((1,), (0,)), ((), ())),
                            preferred_element_type=f32)
    out_ref[0] = jnp.maximum(h + b3_ref[...], 0.0)


@jax.jit
def kernel(unknown, known, unknow_feats, known_feats, W1, b1, W2, b2, W3, b3):
    B, n, _ = unknown.shape
    m = known.shape[1]
    C1 = unknow_feats.shape[1]
    C2 = known_feats.shape[1]
    O = W3.shape[0]
    nb = _NB
    grid = (B, n // nb)

    ut = jnp.transpose(unknown, (0, 2, 1))

    out = pl.pallas_call(
        _fp_kernel,
        grid=grid,
        in_specs=[
            pl.BlockSpec((1, 3, nb), lambda b, j: (b, 0, j)),
            pl.BlockSpec((1, m, 3), lambda b, j: (b, 0, 0)),
            pl.BlockSpec((1, C1, nb), lambda b, j: (b, 0, j)),
            pl.BlockSpec((1, C2, m), lambda b, j: (b, 0, 0)),
            pl.BlockSpec((W1.shape[0], W1.shape[1]), lambda b, j: (0, 0)),
            pl.BlockSpec((W1.shape[0], 1), lambda b, j: (0, 0)),
            pl.BlockSpec((W2.shape[0], W2.shape[1]), lambda b, j: (0, 0)),
            pl.BlockSpec((W2.shape[0], 1), lambda b, j: (0, 0)),
            pl.BlockSpec((W3.shape[0], W3.shape[1]), lambda b, j: (0, 0)),
            pl.BlockSpec((W3.shape[0], 1), lambda b, j: (0, 0)),
        ],
        out_specs=pl.BlockSpec((1, O, nb), lambda b, j: (b, 0, j)),
        out_shape=jax.ShapeDtypeStruct((B, O, n), jnp.float32),
    )(ut, known, unknow_feats, known_feats,
      W1, b1.reshape(-1, 1), W2, b2.reshape(-1, 1), W3, b3.reshape(-1, 1))
    return out
